# Initial kernel scaffold; baseline (speedup 1.0000x reference)
import jax
import jax.numpy as jnp
from jax import lax
from jax.experimental import pallas as pl
from jax.experimental.pallas import tpu as pltpu
from jax.experimental.pallas import tpu_sc as plsc

N = 500000
D = 128
B = 16
C = 200
NUM_CHUNKS = N // C
LANES = 16


def _gml_kernel(feat_hbm, idx_hbm, vec_hbm, out_hbm, fbuf, obuf, ibuf, vbuf):
    nc = 2
    ns = 16
    nw = nc * ns
    wid = lax.axis_index("s") * nc + lax.axis_index("c")

    pltpu.sync_copy(vec_hbm, vbuf)

    lane = lax.iota(jnp.int32, LANES)
    n_my_chunks = (NUM_CHUNKS - wid + nw - 1) // nw

    def chunk_body(k, _):
        c = wid + k * nw
        row0 = c * C
        e0 = row0 * D
        pltpu.sync_copy(feat_hbm.at[pl.ds(e0, C * D)], fbuf)
        pltpu.sync_copy(idx_hbm.at[pl.ds(row0, C)], ibuf)

        def row_body(r, _):
            rsplat = jnp.full((LANES,), r, jnp.int32)
            idxv = plsc.load_gather(ibuf, [rsplat])
            mbase = idxv * D + lane
            fbase = r * D
            for j in range(D // LANES):
                mult = plsc.load_gather(vbuf, [mbase + (LANES * j)])
                f = fbuf[pl.ds(fbase + LANES * j, LANES)]
                obuf[pl.ds(fbase + LANES * j, LANES)] = f * mult
            return 0

        lax.fori_loop(0, C, row_body, 0, unroll=False)
        pltpu.sync_copy(obuf, out_hbm.at[pl.ds(e0, C * D)])
        return 0

    lax.fori_loop(0, n_my_chunks, chunk_body, 0, unroll=False)


@jax.jit
def _gml(features_flat, point_idx, vecter_flat):
    mesh = plsc.VectorSubcoreMesh(core_axis_name="c", subcore_axis_name="s")
    run = pl.kernel(
        _gml_kernel,
        out_type=jax.ShapeDtypeStruct((N * D,), jnp.float32),
        mesh=mesh,
        scratch_types=[
            pltpu.VMEM((C * D,), jnp.float32),
            pltpu.VMEM((C * D,), jnp.float32),
            pltpu.VMEM((C,), jnp.int32),
            pltpu.VMEM((B * D,), jnp.float32),
        ],
    )
    return run(features_flat, point_idx, vecter_flat)


def kernel(features, point_idx, vecter):
    out = _gml(
        features.reshape(N * D),
        point_idx.astype(jnp.int32),
        vecter.reshape(B * D),
    )
    return out.reshape(N, D)


# speedup vs baseline: 1.1082x; 1.1082x over previous
import jax
import jax.numpy as jnp
from jax import lax
from jax.experimental import pallas as pl
from jax.experimental.pallas import tpu as pltpu
from jax.experimental.pallas import tpu_sc as plsc

N = 500000
D = 128
B = 16
C = 200
NUM_CHUNKS = N // C
LANES = 16


def _gml_kernel(feat_hbm, idx_hbm, vec_hbm, out_hbm, fbuf, obuf, ibuf, vbuf):
    nc = 2
    ns = 16
    nw = nc * ns
    wid = lax.axis_index("s") * nc + lax.axis_index("c")

    pltpu.sync_copy(vec_hbm, vbuf)

    lane = lax.iota(jnp.int32, LANES)
    n_my_chunks = (NUM_CHUNKS - wid + nw - 1) // nw

    def chunk_body(k, _):
        c = wid + k * nw
        row0 = c * C
        e0 = row0 * D
        pltpu.sync_copy(feat_hbm.at[pl.ds(e0, C * D)], fbuf)
        pltpu.sync_copy(idx_hbm.at[pl.ds(row0, C)], ibuf)

        def row_body(r, _):
            rsplat = jnp.full((LANES,), r, jnp.int32)
            idxv = plsc.load_gather(ibuf, [rsplat])
            mbase = idxv * D + lane
            fbase = r * D
            for j in range(D // LANES):
                mult = plsc.load_gather(vbuf, [mbase + (LANES * j)])
                f = fbuf[pl.ds(fbase + LANES * j, LANES)]
                obuf[pl.ds(fbase + LANES * j, LANES)] = f * mult
            return 0

        lax.fori_loop(0, C, row_body, 0, unroll=False)
        pltpu.sync_copy(obuf, out_hbm.at[pl.ds(e0, C * D)])
        return 0

    lax.fori_loop(0, n_my_chunks, chunk_body, 0, unroll=False)


@jax.jit
def _gml(features_flat, point_idx, vecter_flat):
    mesh = plsc.VectorSubcoreMesh(core_axis_name="c", subcore_axis_name="s")
    run = pl.kernel(
        _gml_kernel,
        out_type=jax.ShapeDtypeStruct((N * D,), jnp.float32),
        mesh=mesh,
        scratch_types=[
            pltpu.VMEM((C * D,), jnp.float32),
            pltpu.VMEM((C * D,), jnp.float32),
            pltpu.VMEM((C,), jnp.int32),
            pltpu.VMEM((B * D,), jnp.float32),
        ],
        compiler_params=pltpu.CompilerParams(needs_layout_passes=False),
    )
    return run(features_flat, point_idx, vecter_flat)


def kernel(features, point_idx, vecter):
    out = _gml(
        features.reshape(N * D),
        point_idx.astype(jnp.int32),
        vecter.reshape(B * D),
    )
    return out.reshape(N, D)
